# baseline (speedup 1.0000x reference)
import functools

import jax
import jax.numpy as jnp
from jax import lax
from jax.experimental import pallas as pl
from jax.experimental.pallas import tpu as pltpu
from jax.experimental.pallas import tpu_sc as plsc

NCORE = 2
NSUB = 16
NW = NCORE * NSUB

_GDN = lax.GatherDimensionNumbers(
    offset_dims=(), collapsed_slice_dims=(0,), start_index_map=(0,))


def _lane_gather(v16, idx16):
    return lax.gather(v16, idx16[:, None], dimension_numbers=_GDN,
                      slice_sizes=(1,),
                      mode=lax.GatherScatterMode.PROMISE_IN_BOUNDS)


def _tc1_body(x_ref, w1_ref, acatA_ref, acatB_ref, h_ref, tA_ref, tB_ref):
    h = jnp.dot(x_ref[...], w1_ref[...], preferred_element_type=jnp.float32)
    h_ref[...] = h
    tA_ref[...] = jnp.dot(h, acatA_ref[...], preferred_element_type=jnp.float32)
    tB_ref[...] = jnp.dot(h, acatB_ref[...], preferred_element_type=jnp.float32)


def _tc2_body(p0_ref, p1_ref, d0_ref, d1_ref, r16_ref, b1_ref, w2_ref,
              a2A_ref, a2B_ref, h2_ref, t2A_ref, t2B_ref):
    den = jnp.dot(d0_ref[...] + d1_ref[...], r16_ref[...],
                  preferred_element_type=jnp.float32)
    out1 = (p0_ref[...] + p1_ref[...]) / (den + 1e-9) + b1_ref[...]
    hmid = jnp.where(out1 > 0, out1, jnp.exp(jnp.minimum(out1, 0.0)) - 1.0)
    h2 = jnp.dot(hmid, w2_ref[...], preferred_element_type=jnp.float32)
    h2_ref[...] = h2
    t2A_ref[...] = jnp.dot(h2, a2A_ref[...], preferred_element_type=jnp.float32)
    t2B_ref[...] = jnp.dot(h2, a2B_ref[...], preferred_element_type=jnp.float32)


def _tc3_body(p0_ref, p1_ref, d0_ref, d1_ref, r2_ref, b2_ref, out_ref):
    den = jnp.dot(d0_ref[...] + d1_ref[...], r2_ref[...],
                  preferred_element_type=jnp.float32)
    out_ref[...] = (p0_ref[...] + p1_ref[...]) / (den + 1e-9) + b2_ref[...]


def _make_sc_edge_kernel(n_pad, e_pad, feat, chunk, heads):
    ept = e_pad // NW
    nchunk = ept // chunk
    assert nchunk % 2 == 0 and chunk % 16 == 0
    rows_per_sub = n_pad // NSUB
    width = feat + 16
    nheads_blk = feat // 16

    mesh = plsc.VectorSubcoreMesh(core_axis_name="c", subcore_axis_name="s")

    @functools.partial(
        pl.kernel,
        mesh=mesh,
        compiler_params=pltpu.CompilerParams(use_tc_tiling_on_sc=False),
        out_type=[
            jax.ShapeDtypeStruct((NCORE * n_pad, feat), jnp.float32),
            jax.ShapeDtypeStruct((NCORE * n_pad, 16), jnp.float32),
        ],
        scratch_types=[
            pltpu.VMEM((nchunk, chunk), jnp.int32),
            pltpu.VMEM((2, chunk), jnp.int32),
            pltpu.VMEM((2, chunk), jnp.int32),
            pltpu.VMEM((2, chunk, 16), jnp.float32),
            pltpu.VMEM((2, chunk, 16), jnp.float32),
            pltpu.VMEM((2, chunk, feat), jnp.float32),
            pltpu.VMEM((2, chunk, 16), jnp.float32),
            pltpu.VMEM_SHARED((n_pad, feat), jnp.float32),
            pltpu.VMEM_SHARED((n_pad, 16), jnp.float32),
            pltpu.SemaphoreType.DMA,
            pltpu.SemaphoreType.DMA,
            pltpu.SemaphoreType.DMA,
            pltpu.SemaphoreType.DMA,
        ],
    )
    def sc_kernel(pk_hbm, tabA_hbm, tabB_hbm, h_hbm, zf_hbm, z16_hbm,
                  outp_hbm, denp_hbm,
                  pkidx, sidx, didx, tabs, tabd, hb, exb, out_sh, den_sh,
                  g0, g1, s0, s1):
        c = lax.axis_index("c")
        s = lax.axis_index("s")
        wid = c * NSUB + s
        rbase = s * rows_per_sub
        gsem = (g0, g1)
        ssem = (s0, s1)

        pltpu.sync_copy(zf_hbm.at[pl.ds(rbase, rows_per_sub)],
                        out_sh.at[pl.ds(rbase, rows_per_sub)])
        pltpu.sync_copy(z16_hbm.at[pl.ds(rbase, rows_per_sub)],
                        den_sh.at[pl.ds(rbase, rows_per_sub)])
        plsc.subcore_barrier()

        pltpu.sync_copy(pk_hbm.at[pl.ds(wid * nchunk, nchunk)], pkidx)

        def unpack(k, p):
            @pl.loop(0, chunk // 16)
            def _(g):
                pk = pkidx[k, pl.ds(g * 16, 16)]
                sidx[p, pl.ds(g * 16, 16)] = pk & 0x3FFF
                didx[p, pl.ds(g * 16, 16)] = pk >> 14

        def issue_gathers(p):
            pltpu.async_copy(tabA_hbm.at[sidx.at[p]], tabs.at[p], gsem[p])
            pltpu.async_copy(tabB_hbm.at[didx.at[p]], tabd.at[p], gsem[p])
            pltpu.async_copy(h_hbm.at[sidx.at[p]], hb.at[p], gsem[p])

        def wait_gathers(p):
            pltpu.make_async_copy(tabA_hbm.at[sidx.at[p]], tabs.at[p],
                                  gsem[p]).wait()
            pltpu.make_async_copy(tabB_hbm.at[didx.at[p]], tabd.at[p],
                                  gsem[p]).wait()
            pltpu.make_async_copy(h_hbm.at[sidx.at[p]], hb.at[p],
                                  gsem[p]).wait()

        def issue_scatter(p):
            pltpu.async_copy(hb.at[p], out_sh.at[didx.at[p]], ssem[p],
                             add=True)
            pltpu.async_copy(exb.at[p], den_sh.at[didx.at[p]], ssem[p],
                             add=True)

        def wait_scatter(p):
            pltpu.make_async_copy(hb.at[p], out_sh.at[didx.at[p]],
                                  ssem[p]).wait()
            pltpu.make_async_copy(exb.at[p], den_sh.at[didx.at[p]],
                                  ssem[p]).wait()

        def compute(p):
            tabs_p, tabd_p, hx_p, exb_p = (tabs.at[p], tabd.at[p], hb.at[p],
                                           exb.at[p])

            @pl.loop(0, chunk)
            def _edge(j):
                e = tabs_p[j, :] + tabd_p[j, :]
                e = jnp.maximum(e, 0.2 * e)
                ex = jnp.exp(e)
                exb_p[j, :] = ex
                for h in range(heads):
                    idx = jnp.full((16,), h, dtype=jnp.int32)
                    bc = _lane_gather(ex, idx)
                    if heads == 1:
                        for fblk in range(nheads_blk):
                            slc = (j, pl.ds(fblk * 16, 16))
                            hx_p[slc] = hx_p[slc] * bc
                    else:
                        slc = (j, pl.ds(h * 16, 16))
                        hx_p[slc] = hx_p[slc] * bc

        unpack(0, 0)
        issue_gathers(0)

        @pl.loop(0, nchunk // 2)
        def _pair(i):
            for p in (0, 1):
                kk = 2 * i + p
                wait_gathers(p)

                @pl.when(kk + 1 < nchunk)
                def _prefetch():
                    @pl.when(kk >= 1)
                    def _drain():
                        wait_scatter(1 - p)
                    unpack(kk + 1, 1 - p)
                    issue_gathers(1 - p)

                compute(p)
                issue_scatter(p)

        wait_scatter(0)
        wait_scatter(1)
        plsc.subcore_barrier()
        pltpu.sync_copy(out_sh.at[pl.ds(rbase, rows_per_sub)],
                        outp_hbm.at[pl.ds(c * n_pad + rbase, rows_per_sub)])
        pltpu.sync_copy(den_sh.at[pl.ds(rbase, rows_per_sub)],
                        denp_hbm.at[pl.ds(c * n_pad + rbase, rows_per_sub)])

    return sc_kernel


def _make_sc_take_kernel(n_nodes, feat, nb):
    rows_per_tile = nb // NW
    mesh = plsc.VectorSubcoreMesh(core_axis_name="c", subcore_axis_name="s")

    @functools.partial(
        pl.kernel,
        mesh=mesh,
        compiler_params=pltpu.CompilerParams(use_tc_tiling_on_sc=False),
        out_type=jax.ShapeDtypeStruct((nb, feat), jnp.float32),
        scratch_types=[
            pltpu.VMEM((rows_per_tile,), jnp.int32),
            pltpu.VMEM((rows_per_tile, feat), jnp.float32),
        ],
    )
    def take_kernel(tab_hbm, idx_hbm, out_hbm, idxv, rowsv):
        c = lax.axis_index("c")
        s = lax.axis_index("s")
        wid = c * NSUB + s
        base = wid * rows_per_tile
        pltpu.sync_copy(idx_hbm.at[pl.ds(base, rows_per_tile)], idxv)
        pltpu.sync_copy(tab_hbm.at[idxv], rowsv)
        pltpu.sync_copy(rowsv, out_hbm.at[pl.ds(base, rows_per_tile)])

    return take_kernel


CHUNK1 = 80
CHUNK2 = 128


@jax.jit
def kernel(x, edge_index, node_index, W1, a1_src, a1_dst, b1,
           W2, a2_src, a2_dst, b2):
    n, f_in = x.shape
    e = edge_index.shape[1]
    heads, hid = a1_src.shape
    out_f = W2.shape[1]
    nb = node_index.shape[0]
    hdim = heads * hid

    import math
    n_pad = ((n + 8 * NSUB - 1) // (8 * NSUB)) * (8 * NSUB)
    egran = NW * 2 * CHUNK1 * CHUNK2 // math.gcd(CHUNK1, CHUNK2)
    e_pad = ((e + egran - 1) // egran) * egran

    src = edge_index[0]
    dst = edge_index[1]
    pad_e = e_pad - e
    pad_dst = (n + jnp.arange(pad_e, dtype=jnp.int32) % (n_pad - n)
               ).astype(jnp.int32)
    src_p = jnp.concatenate([src, jnp.zeros((pad_e,), jnp.int32)])
    dst_p = jnp.concatenate([dst, pad_dst])
    packed = src_p | (dst_p << 14)

    eyeh = jnp.eye(heads, dtype=jnp.float32)
    A1s = (eyeh[:, None, :] * a1_src[:, :, None]).reshape(hdim, heads)
    A1d = (eyeh[:, None, :] * a1_dst[:, :, None]).reshape(hdim, heads)
    acatA = jnp.concatenate([A1s, A1d], axis=1)
    acatB = jnp.concatenate([A1d, A1s], axis=1)
    pad2 = jnp.zeros((out_f, 16 - 2), jnp.float32)
    a2A = jnp.concatenate([a2_src.T, a2_dst.T, pad2], axis=1)
    a2B = jnp.concatenate([a2_dst.T, a2_src.T, pad2], axis=1)

    ids = jax.lax.broadcasted_iota(jnp.int32, (16, hdim), 0)
    cols = jax.lax.broadcasted_iota(jnp.int32, (16, hdim), 1)
    R16 = jnp.where(ids == cols // hid, 1.0, 0.0).astype(jnp.float32)
    R2 = jnp.where(jax.lax.broadcasted_iota(jnp.int32, (16, out_f), 0) == 0,
                   1.0, 0.0).astype(jnp.float32)

    blk = 1000
    grid = (n // blk,)

    def full(shape):
        return pl.BlockSpec(shape, lambda i: (0, 0))

    def rows(width):
        return pl.BlockSpec((blk, width), lambda i: (i, 0))

    h1, tabA, tabB = pl.pallas_call(
        _tc1_body,
        grid=grid,
        in_specs=[rows(f_in), full((f_in, hdim)), full((hdim, 16)),
                  full((hdim, 16))],
        out_specs=[rows(hdim), rows(16), rows(16)],
        out_shape=[jax.ShapeDtypeStruct((n, hdim), jnp.float32),
                   jax.ShapeDtypeStruct((n, 16), jnp.float32),
                   jax.ShapeDtypeStruct((n, 16), jnp.float32)],
    )(x, W1, acatA, acatB)

    tabBp = jnp.concatenate(
        [tabB, jnp.zeros((n_pad - n, 16), jnp.float32)], axis=0)

    zf = jnp.zeros((n_pad, hdim), jnp.float32)
    z16 = jnp.zeros((n_pad, 16), jnp.float32)
    sc1 = _make_sc_edge_kernel(n_pad, e_pad, hdim, CHUNK1, heads)
    outp1, denp1 = sc1(packed.reshape(-1, CHUNK1), tabA, tabBp, h1, zf, z16)

    h2, tab2A, tab2B = pl.pallas_call(
        _tc2_body,
        grid=grid,
        in_specs=[rows(hdim), rows(hdim), rows(16), rows(16),
                  full((16, hdim)), pl.BlockSpec((1, hdim), lambda i: (0, 0)),
                  full((hdim, out_f)), full((out_f, 16)), full((out_f, 16))],
        out_specs=[rows(out_f), rows(16), rows(16)],
        out_shape=[jax.ShapeDtypeStruct((n, out_f), jnp.float32),
                   jax.ShapeDtypeStruct((n, 16), jnp.float32),
                   jax.ShapeDtypeStruct((n, 16), jnp.float32)],
    )(outp1[:n], outp1[n_pad:n_pad + n], denp1[:n], denp1[n_pad:n_pad + n],
      R16, b1.reshape(1, hdim), W2, a2A, a2B)

    tab2Bp = jnp.concatenate(
        [tab2B, jnp.zeros((n_pad - n, 16), jnp.float32)], axis=0)

    zf2 = jnp.zeros((n_pad, out_f), jnp.float32)
    sc2 = _make_sc_edge_kernel(n_pad, e_pad, out_f, CHUNK2, 1)
    outp2, denp2 = sc2(packed.reshape(-1, CHUNK2), tab2A, tab2Bp, h2, zf2, z16)

    out2 = pl.pallas_call(
        _tc3_body,
        grid=grid,
        in_specs=[rows(out_f), rows(out_f), rows(16), rows(16),
                  full((16, out_f)), pl.BlockSpec((1, out_f), lambda i: (0, 0))],
        out_specs=rows(out_f),
        out_shape=jax.ShapeDtypeStruct((n, out_f), jnp.float32),
    )(outp2[:n], outp2[n_pad:n_pad + n], denp2[:n], denp2[n_pad:n_pad + n],
      R2, b2.reshape(1, out_f))

    take = _make_sc_take_kernel(n, out_f, nb)
    return take(out2, node_index)


# speedup vs baseline: 1.5549x; 1.5549x over previous
import functools

import jax
import jax.numpy as jnp
from jax import lax
from jax.experimental import pallas as pl
from jax.experimental.pallas import tpu as pltpu
from jax.experimental.pallas import tpu_sc as plsc

NCORE = 2
NSUB = 16
NW = NCORE * NSUB

_GDN = lax.GatherDimensionNumbers(
    offset_dims=(), collapsed_slice_dims=(0,), start_index_map=(0,))


def _lane_gather(v16, idx16):
    return lax.gather(v16, idx16[:, None], dimension_numbers=_GDN,
                      slice_sizes=(1,),
                      mode=lax.GatherScatterMode.PROMISE_IN_BOUNDS)


def _tc1_body(x_ref, w1_ref, acatA_ref, acatB_ref, h_ref, tA_ref, tB_ref):
    h = jnp.dot(x_ref[...], w1_ref[...], preferred_element_type=jnp.float32)
    h_ref[...] = h
    tA_ref[...] = jnp.dot(h, acatA_ref[...], preferred_element_type=jnp.float32)
    tB_ref[...] = jnp.dot(h, acatB_ref[...], preferred_element_type=jnp.float32)


def _tc2_body(p0_ref, p1_ref, d0_ref, d1_ref, r16_ref, b1_ref, w2_ref,
              a2A_ref, a2B_ref, h2_ref, t2A_ref, t2B_ref):
    den = jnp.dot(d0_ref[...] + d1_ref[...], r16_ref[...],
                  preferred_element_type=jnp.float32)
    out1 = (p0_ref[...] + p1_ref[...]) / (den + 1e-9) + b1_ref[...]
    hmid = jnp.where(out1 > 0, out1, jnp.exp(jnp.minimum(out1, 0.0)) - 1.0)
    h2 = jnp.dot(hmid, w2_ref[...], preferred_element_type=jnp.float32)
    h2_ref[...] = h2
    t2A_ref[...] = jnp.dot(h2, a2A_ref[...], preferred_element_type=jnp.float32)
    t2B_ref[...] = jnp.dot(h2, a2B_ref[...], preferred_element_type=jnp.float32)


def _tc3_body(p0_ref, p1_ref, d0_ref, d1_ref, r2_ref, b2_ref, out_ref):
    den = jnp.dot(d0_ref[...] + d1_ref[...], r2_ref[...],
                  preferred_element_type=jnp.float32)
    out_ref[...] = (p0_ref[...] + p1_ref[...]) / (den + 1e-9) + b2_ref[...]


def _make_sc_edge_kernel(n_pad, e_pad, feat, chunk, heads):
    ept = e_pad // NW
    nchunk = ept // chunk
    assert nchunk % 2 == 0 and chunk % 16 == 0
    rows_per_sub = n_pad // NSUB
    width = feat + 16
    nheads_blk = feat // 16

    mesh = plsc.VectorSubcoreMesh(core_axis_name="c", subcore_axis_name="s")

    @functools.partial(
        pl.kernel,
        mesh=mesh,
        compiler_params=pltpu.CompilerParams(use_tc_tiling_on_sc=False),
        out_type=[
            jax.ShapeDtypeStruct((NCORE * n_pad, feat), jnp.float32),
            jax.ShapeDtypeStruct((NCORE * n_pad, 16), jnp.float32),
        ],
        scratch_types=[
            pltpu.VMEM((nchunk, chunk), jnp.int32),
            pltpu.VMEM((2, chunk), jnp.int32),
            pltpu.VMEM((2, chunk), jnp.int32),
            pltpu.VMEM((2, chunk, 16), jnp.float32),
            pltpu.VMEM((2, chunk, 16), jnp.float32),
            pltpu.VMEM((2, chunk, feat), jnp.float32),
            pltpu.VMEM((2, chunk, 16), jnp.float32),
            pltpu.VMEM_SHARED((n_pad, feat), jnp.float32),
            pltpu.VMEM_SHARED((n_pad, 16), jnp.float32),
            pltpu.SemaphoreType.DMA,
            pltpu.SemaphoreType.DMA,
            pltpu.SemaphoreType.DMA,
            pltpu.SemaphoreType.DMA,
        ],
    )
    def sc_kernel(pk_hbm, tabA_hbm, tabB_hbm, h_hbm, zf_hbm, z16_hbm,
                  outp_hbm, denp_hbm,
                  pkidx, sidx, didx, tabs, tabd, hb, exb, out_sh, den_sh,
                  g0, g1, s0, s1):
        c = lax.axis_index("c")
        s = lax.axis_index("s")
        wid = c * NSUB + s
        rbase = s * rows_per_sub
        gsem = (g0, g1)
        ssem = (s0, s1)

        pltpu.sync_copy(zf_hbm.at[pl.ds(rbase, rows_per_sub)],
                        out_sh.at[pl.ds(rbase, rows_per_sub)])
        pltpu.sync_copy(z16_hbm.at[pl.ds(rbase, rows_per_sub)],
                        den_sh.at[pl.ds(rbase, rows_per_sub)])
        plsc.subcore_barrier()

        pltpu.sync_copy(pk_hbm.at[pl.ds(wid * nchunk, nchunk)], pkidx)

        def unpack(k, p):
            @pl.loop(0, chunk // 16)
            def _(g):
                pk = pkidx[k, pl.ds(g * 16, 16)]
                sidx[p, pl.ds(g * 16, 16)] = pk & 0x3FFF
                didx[p, pl.ds(g * 16, 16)] = pk >> 14

        def issue_gathers(p):
            pltpu.async_copy(tabA_hbm.at[sidx.at[p]], tabs.at[p], gsem[p])
            pltpu.async_copy(tabB_hbm.at[didx.at[p]], tabd.at[p], gsem[p])
            pltpu.async_copy(h_hbm.at[sidx.at[p]], hb.at[p], gsem[p])

        def wait_gathers(p):
            pltpu.make_async_copy(tabA_hbm.at[sidx.at[p]], tabs.at[p],
                                  gsem[p]).wait()
            pltpu.make_async_copy(tabB_hbm.at[didx.at[p]], tabd.at[p],
                                  gsem[p]).wait()
            pltpu.make_async_copy(h_hbm.at[sidx.at[p]], hb.at[p],
                                  gsem[p]).wait()

        def issue_scatter(p):
            pltpu.async_copy(hb.at[p], out_sh.at[didx.at[p]], ssem[p],
                             add=True)
            pltpu.async_copy(exb.at[p], den_sh.at[didx.at[p]], ssem[p],
                             add=True)

        def wait_scatter(p):
            pltpu.make_async_copy(hb.at[p], out_sh.at[didx.at[p]],
                                  ssem[p]).wait()
            pltpu.make_async_copy(exb.at[p], den_sh.at[didx.at[p]],
                                  ssem[p]).wait()

        def compute(p):
            tabs_p, tabd_p, hx_p, exb_p = (tabs.at[p], tabd.at[p], hb.at[p],
                                           exb.at[p])

            @pl.loop(0, chunk)
            def _edge(j):
                e = tabs_p[j, :] + tabd_p[j, :]
                e = jnp.maximum(e, 0.2 * e)
                ex = jnp.exp(e)
                exb_p[j, :] = ex
                for h in range(heads):
                    idx = jnp.full((16,), h, dtype=jnp.int32)
                    bc = _lane_gather(ex, idx)
                    if heads == 1:
                        for fblk in range(nheads_blk):
                            slc = (j, pl.ds(fblk * 16, 16))
                            hx_p[slc] = hx_p[slc] * bc
                    else:
                        slc = (j, pl.ds(h * 16, 16))
                        hx_p[slc] = hx_p[slc] * bc

        unpack(0, 0)
        issue_gathers(0)

        @pl.loop(0, nchunk // 2)
        def _pair(i):
            for p in (0, 1):
                kk = 2 * i + p
                wait_gathers(p)

                @pl.when(kk + 1 < nchunk)
                def _prefetch():
                    @pl.when(kk >= 1)
                    def _drain():
                        wait_scatter(1 - p)
                    unpack(kk + 1, 1 - p)
                    issue_gathers(1 - p)

                compute(p)
                issue_scatter(p)

        wait_scatter(0)
        wait_scatter(1)
        plsc.subcore_barrier()
        pltpu.sync_copy(out_sh.at[pl.ds(rbase, rows_per_sub)],
                        outp_hbm.at[pl.ds(c * n_pad + rbase, rows_per_sub)])
        pltpu.sync_copy(den_sh.at[pl.ds(rbase, rows_per_sub)],
                        denp_hbm.at[pl.ds(c * n_pad + rbase, rows_per_sub)])

    return sc_kernel


def _make_sc_take_kernel(n_nodes, feat, nb):
    rows_per_tile = nb // NW
    mesh = plsc.VectorSubcoreMesh(core_axis_name="c", subcore_axis_name="s")

    @functools.partial(
        pl.kernel,
        mesh=mesh,
        compiler_params=pltpu.CompilerParams(use_tc_tiling_on_sc=False),
        out_type=jax.ShapeDtypeStruct((nb, feat), jnp.float32),
        scratch_types=[
            pltpu.VMEM((rows_per_tile,), jnp.int32),
            pltpu.VMEM((rows_per_tile, feat), jnp.float32),
        ],
    )
    def take_kernel(tab_hbm, idx_hbm, out_hbm, idxv, rowsv):
        c = lax.axis_index("c")
        s = lax.axis_index("s")
        wid = c * NSUB + s
        base = wid * rows_per_tile
        pltpu.sync_copy(idx_hbm.at[pl.ds(base, rows_per_tile)], idxv)
        pltpu.sync_copy(tab_hbm.at[idxv], rowsv)
        pltpu.sync_copy(rowsv, out_hbm.at[pl.ds(base, rows_per_tile)])

    return take_kernel


CHUNK1 = 80
CHUNK2 = 128


@jax.jit
def kernel(x, edge_index, node_index, W1, a1_src, a1_dst, b1,
           W2, a2_src, a2_dst, b2):
    n, f_in = x.shape
    e = edge_index.shape[1]
    heads, hid = a1_src.shape
    out_f = W2.shape[1]
    nb = node_index.shape[0]
    hdim = heads * hid

    import math
    n_pad = ((n + 8 * NSUB - 1) // (8 * NSUB)) * (8 * NSUB)
    egran = NW * 2 * CHUNK1 * CHUNK2 // math.gcd(CHUNK1, CHUNK2)
    e_pad = ((e + egran - 1) // egran) * egran

    src = edge_index[0]
    dst = edge_index[1]
    pad_e = e_pad - e
    pad_iota = jnp.arange(pad_e, dtype=jnp.int32)
    pad_dst = n + pad_iota % (n_pad - n)
    pad_src = pad_iota % n
    src_p = jnp.concatenate([src, pad_src])
    dst_p = jnp.concatenate([dst, pad_dst])
    packed = src_p | (dst_p << 14)

    eyeh = jnp.eye(heads, dtype=jnp.float32)
    A1s = (eyeh[:, None, :] * a1_src[:, :, None]).reshape(hdim, heads)
    A1d = (eyeh[:, None, :] * a1_dst[:, :, None]).reshape(hdim, heads)
    acatA = jnp.concatenate([A1s, A1d], axis=1)
    acatB = jnp.concatenate([A1d, A1s], axis=1)
    pad2 = jnp.zeros((out_f, 16 - 2), jnp.float32)
    a2A = jnp.concatenate([a2_src.T, a2_dst.T, pad2], axis=1)
    a2B = jnp.concatenate([a2_dst.T, a2_src.T, pad2], axis=1)

    ids = jax.lax.broadcasted_iota(jnp.int32, (16, hdim), 0)
    cols = jax.lax.broadcasted_iota(jnp.int32, (16, hdim), 1)
    R16 = jnp.where(ids == cols // hid, 1.0, 0.0).astype(jnp.float32)
    R2 = jnp.where(jax.lax.broadcasted_iota(jnp.int32, (16, out_f), 0) == 0,
                   1.0, 0.0).astype(jnp.float32)

    blk = 1000
    grid = (n // blk,)

    def full(shape):
        return pl.BlockSpec(shape, lambda i: (0, 0))

    def rows(width):
        return pl.BlockSpec((blk, width), lambda i: (i, 0))

    h1, tabA, tabB = pl.pallas_call(
        _tc1_body,
        grid=grid,
        in_specs=[rows(f_in), full((f_in, hdim)), full((hdim, 16)),
                  full((hdim, 16))],
        out_specs=[rows(hdim), rows(16), rows(16)],
        out_shape=[jax.ShapeDtypeStruct((n, hdim), jnp.float32),
                   jax.ShapeDtypeStruct((n, 16), jnp.float32),
                   jax.ShapeDtypeStruct((n, 16), jnp.float32)],
    )(x, W1, acatA, acatB)

    tabBp = jnp.concatenate(
        [tabB, jnp.zeros((n_pad - n, 16), jnp.float32)], axis=0)

    zf = jnp.zeros((n_pad, hdim), jnp.float32)
    z16 = jnp.zeros((n_pad, 16), jnp.float32)
    sc1 = _make_sc_edge_kernel(n_pad, e_pad, hdim, CHUNK1, heads)
    outp1, denp1 = sc1(packed.reshape(-1, CHUNK1), tabA, tabBp, h1, zf, z16)

    h2, tab2A, tab2B = pl.pallas_call(
        _tc2_body,
        grid=grid,
        in_specs=[rows(hdim), rows(hdim), rows(16), rows(16),
                  full((16, hdim)), pl.BlockSpec((1, hdim), lambda i: (0, 0)),
                  full((hdim, out_f)), full((out_f, 16)), full((out_f, 16))],
        out_specs=[rows(out_f), rows(16), rows(16)],
        out_shape=[jax.ShapeDtypeStruct((n, out_f), jnp.float32),
                   jax.ShapeDtypeStruct((n, 16), jnp.float32),
                   jax.ShapeDtypeStruct((n, 16), jnp.float32)],
    )(outp1[:n], outp1[n_pad:n_pad + n], denp1[:n], denp1[n_pad:n_pad + n],
      R16, b1.reshape(1, hdim), W2, a2A, a2B)

    tab2Bp = jnp.concatenate(
        [tab2B, jnp.zeros((n_pad - n, 16), jnp.float32)], axis=0)

    zf2 = jnp.zeros((n_pad, out_f), jnp.float32)
    sc2 = _make_sc_edge_kernel(n_pad, e_pad, out_f, CHUNK2, 1)
    outp2, denp2 = sc2(packed.reshape(-1, CHUNK2), tab2A, tab2Bp, h2, zf2, z16)

    out2 = pl.pallas_call(
        _tc3_body,
        grid=grid,
        in_specs=[rows(out_f), rows(out_f), rows(16), rows(16),
                  full((16, out_f)), pl.BlockSpec((1, out_f), lambda i: (0, 0))],
        out_specs=rows(out_f),
        out_shape=jax.ShapeDtypeStruct((n, out_f), jnp.float32),
    )(outp2[:n], outp2[n_pad:n_pad + n], denp2[:n], denp2[n_pad:n_pad + n],
      R2, b2.reshape(1, out_f))

    take = _make_sc_take_kernel(n, out_f, nb)
    return take(out2, node_index)


# speedup vs baseline: 1.6690x; 1.0734x over previous
import functools
import math

import jax
import jax.numpy as jnp
from jax import lax
from jax.experimental import pallas as pl
from jax.experimental.pallas import tpu as pltpu
from jax.experimental.pallas import tpu_sc as plsc

NCORE = 2
NSUB = 16
NW = NCORE * NSUB

_GDN = lax.GatherDimensionNumbers(
    offset_dims=(), collapsed_slice_dims=(0,), start_index_map=(0,))


def _lane_gather(v16, idx16):
    return lax.gather(v16, idx16[:, None], dimension_numbers=_GDN,
                      slice_sizes=(1,),
                      mode=lax.GatherScatterMode.PROMISE_IN_BOUNDS)


def _tc1_body(x_ref, w1_ref, acatA_ref, acatB_ref, hx_ref, tB_ref):
    feat = w1_ref.shape[1]
    h = jnp.dot(x_ref[...], w1_ref[...], preferred_element_type=jnp.float32)
    hx_ref[:, :feat] = h
    hx_ref[:, feat:] = jnp.dot(h, acatA_ref[...],
                               preferred_element_type=jnp.float32)
    tB_ref[...] = jnp.dot(h, acatB_ref[...], preferred_element_type=jnp.float32)


def _tc2_body(p0_ref, p1_ref, r16_ref, b1_ref, w2_ref,
              a2A_ref, a2B_ref, h2x_ref, t2B_ref):
    feat = b1_ref.shape[1]
    out_f = w2_ref.shape[1]
    acc = p0_ref[...] + p1_ref[...]
    den = jnp.dot(acc[:, feat:feat + 16], r16_ref[...],
                  preferred_element_type=jnp.float32)
    out1 = acc[:, :feat] / (den + 1e-9) + b1_ref[...]
    hmid = jnp.where(out1 > 0, out1, jnp.exp(jnp.minimum(out1, 0.0)) - 1.0)
    h2 = jnp.dot(hmid, w2_ref[...], preferred_element_type=jnp.float32)
    h2x_ref[:, :out_f] = h2
    h2x_ref[:, out_f:] = jnp.dot(h2, a2A_ref[...],
                                 preferred_element_type=jnp.float32)
    t2B_ref[...] = jnp.dot(h2, a2B_ref[...], preferred_element_type=jnp.float32)


def _make_sc_edge_kernel(n_pad, e_pad, feat, chunk, heads):
    ept = e_pad // NW
    nchunk = ept // chunk
    assert nchunk % 2 == 0 and chunk % 16 == 0 and chunk <= 128
    rows_per_sub = n_pad // NSUB
    width = feat + 16
    nheads_blk = feat // 16

    mesh = plsc.VectorSubcoreMesh(core_axis_name="c", subcore_axis_name="s")

    @functools.partial(
        pl.kernel,
        mesh=mesh,
        compiler_params=pltpu.CompilerParams(use_tc_tiling_on_sc=False),
        out_type=jax.ShapeDtypeStruct((NCORE * n_pad, width), jnp.float32),
        scratch_types=[
            pltpu.VMEM((nchunk, chunk), jnp.int32),
            pltpu.VMEM((2, chunk), jnp.int32),
            pltpu.VMEM((2, chunk), jnp.int32),
            pltpu.VMEM((2, chunk, 16), jnp.float32),
            pltpu.VMEM((2, chunk, width), jnp.float32),
            pltpu.VMEM_SHARED((n_pad, width), jnp.float32),
            pltpu.SemaphoreType.DMA,
            pltpu.SemaphoreType.DMA,
            pltpu.SemaphoreType.DMA,
            pltpu.SemaphoreType.DMA,
        ],
    )
    def sc_kernel(pk_hbm, hxt_hbm, tabB_hbm, z_hbm, outp_hbm,
                  pkidx, sidx, didx, tabd, hx, out_sh, g0, g1, s0, s1):
        c = lax.axis_index("c")
        s = lax.axis_index("s")
        wid = c * NSUB + s
        rbase = s * rows_per_sub
        gsem = (g0, g1)
        ssem = (s0, s1)

        pltpu.sync_copy(z_hbm.at[pl.ds(rbase, rows_per_sub)],
                        out_sh.at[pl.ds(rbase, rows_per_sub)])
        plsc.subcore_barrier()

        pltpu.sync_copy(pk_hbm.at[pl.ds(wid * nchunk, nchunk)], pkidx)

        def unpack(k, p):
            @pl.loop(0, chunk // 16)
            def _(g):
                pk = pkidx[k, pl.ds(g * 16, 16)]
                sidx[p, pl.ds(g * 16, 16)] = pk & 0x3FFF
                didx[p, pl.ds(g * 16, 16)] = pk >> 14

        def issue_gathers(p):
            pltpu.async_copy(hxt_hbm.at[sidx.at[p]], hx.at[p], gsem[p])
            pltpu.async_copy(tabB_hbm.at[didx.at[p]], tabd.at[p], gsem[p])

        def wait_gathers(p):
            pltpu.make_async_copy(hxt_hbm.at[sidx.at[p]], hx.at[p],
                                  gsem[p]).wait()
            pltpu.make_async_copy(tabB_hbm.at[didx.at[p]], tabd.at[p],
                                  gsem[p]).wait()

        def issue_scatter(p):
            pltpu.async_copy(hx.at[p], out_sh.at[didx.at[p]], ssem[p],
                             add=True)

        def wait_scatter(p):
            pltpu.make_async_copy(hx.at[p], out_sh.at[didx.at[p]],
                                  ssem[p]).wait()

        def compute(p):
            tabd_p, hx_p = tabd.at[p], hx.at[p]

            @pl.loop(0, chunk)
            def _edge(j):
                e = hx_p[j, pl.ds(feat, 16)] + tabd_p[j, :]
                e = jnp.maximum(e, 0.2 * e)
                ex = jnp.exp(e)
                hx_p[j, pl.ds(feat, 16)] = ex
                for h in range(heads):
                    idx = jnp.full((16,), h, dtype=jnp.int32)
                    bc = _lane_gather(ex, idx)
                    if heads == 1:
                        for fblk in range(nheads_blk):
                            slc = (j, pl.ds(fblk * 16, 16))
                            hx_p[slc] = hx_p[slc] * bc
                    else:
                        slc = (j, pl.ds(h * 16, 16))
                        hx_p[slc] = hx_p[slc] * bc

        unpack(0, 0)
        issue_gathers(0)

        @pl.loop(0, nchunk // 2)
        def _pair(i):
            for p in (0, 1):
                kk = 2 * i + p
                wait_gathers(p)

                @pl.when(kk + 1 < nchunk)
                def _prefetch():
                    @pl.when(kk >= 1)
                    def _drain():
                        wait_scatter(1 - p)
                    unpack(kk + 1, 1 - p)
                    issue_gathers(1 - p)

                compute(p)
                issue_scatter(p)

        wait_scatter(0)
        wait_scatter(1)
        plsc.subcore_barrier()
        pltpu.sync_copy(out_sh.at[pl.ds(rbase, rows_per_sub)],
                        outp_hbm.at[pl.ds(c * n_pad + rbase, rows_per_sub)])

    return sc_kernel


def _make_sc_final_kernel(n_pad, out_f, nb):
    rows_per_tile = nb // NW
    width = out_f + 16
    nfb = out_f // 16
    mesh = plsc.VectorSubcoreMesh(core_axis_name="c", subcore_axis_name="s")

    @functools.partial(
        pl.kernel,
        mesh=mesh,
        compiler_params=pltpu.CompilerParams(use_tc_tiling_on_sc=False),
        out_type=jax.ShapeDtypeStruct((nb, out_f), jnp.float32),
        scratch_types=[
            pltpu.VMEM((rows_per_tile,), jnp.int32),
            pltpu.VMEM((rows_per_tile,), jnp.int32),
            pltpu.VMEM((rows_per_tile, width), jnp.float32),
            pltpu.VMEM((rows_per_tile, width), jnp.float32),
            pltpu.VMEM((rows_per_tile, out_f), jnp.float32),
            pltpu.VMEM((out_f,), jnp.float32),
        ],
    )
    def final_kernel(outp_hbm, idx_hbm, b2_hbm, out_hbm,
                     idxv, idxv2, r0, r1, res, bias):
        c = lax.axis_index("c")
        s = lax.axis_index("s")
        wid = c * NSUB + s
        base = wid * rows_per_tile
        pltpu.sync_copy(idx_hbm.at[pl.ds(base, rows_per_tile)], idxv)
        pltpu.sync_copy(b2_hbm, bias)

        @pl.loop(0, rows_per_tile // 16)
        def _(g):
            idxv2[pl.ds(g * 16, 16)] = idxv[pl.ds(g * 16, 16)] + n_pad

        pltpu.sync_copy(outp_hbm.at[idxv], r0)
        pltpu.sync_copy(outp_hbm.at[idxv2], r1)

        zero16 = jnp.full((16,), 0, dtype=jnp.int32)

        @pl.loop(0, rows_per_tile)
        def _row(j):
            den = r0[j, pl.ds(out_f, 16)] + r1[j, pl.ds(out_f, 16)]
            db = _lane_gather(den, zero16) + 1e-9
            for fb in range(nfb):
                slc = (j, pl.ds(fb * 16, 16))
                res[slc] = (r0[slc] + r1[slc]) / db + bias[pl.ds(fb * 16, 16)]

        pltpu.sync_copy(res, out_hbm.at[pl.ds(base, rows_per_tile)])

    return final_kernel


CHUNK1 = 80
CHUNK2 = 128


@jax.jit
def kernel(x, edge_index, node_index, W1, a1_src, a1_dst, b1,
           W2, a2_src, a2_dst, b2):
    n, f_in = x.shape
    e = edge_index.shape[1]
    heads, hid = a1_src.shape
    out_f = W2.shape[1]
    nb = node_index.shape[0]
    hdim = heads * hid

    n_pad = ((n + 8 * NSUB - 1) // (8 * NSUB)) * (8 * NSUB)
    egran = NW * 2 * CHUNK1 * CHUNK2 // math.gcd(CHUNK1, CHUNK2)
    e_pad = ((e + egran - 1) // egran) * egran

    src = edge_index[0]
    dst = edge_index[1]
    pad_e = e_pad - e
    pad_iota = jnp.arange(pad_e, dtype=jnp.int32)
    pad_dst = n + pad_iota % (n_pad - n)
    pad_src = pad_iota % n
    src_p = jnp.concatenate([src, pad_src])
    dst_p = jnp.concatenate([dst, pad_dst])
    packed = src_p | (dst_p << 14)

    eyeh = jnp.eye(heads, dtype=jnp.float32)
    A1s = (eyeh[:, None, :] * a1_src[:, :, None]).reshape(hdim, heads)
    A1d = (eyeh[:, None, :] * a1_dst[:, :, None]).reshape(hdim, heads)
    acatA = jnp.concatenate([A1s, A1d], axis=1)
    acatB = jnp.concatenate([A1d, A1s], axis=1)
    pad2 = jnp.zeros((out_f, 16 - 2), jnp.float32)
    a2A = jnp.concatenate([a2_src.T, a2_dst.T, pad2], axis=1)
    a2B = jnp.concatenate([a2_dst.T, a2_src.T, pad2], axis=1)

    ids = jax.lax.broadcasted_iota(jnp.int32, (16, hdim), 0)
    cols = jax.lax.broadcasted_iota(jnp.int32, (16, hdim), 1)
    R16 = jnp.where(ids == cols // hid, 1.0, 0.0).astype(jnp.float32)

    blk = 1000
    grid = (n // blk,)

    def full(shape):
        return pl.BlockSpec(shape, lambda i: (0, 0))

    def rows(width):
        return pl.BlockSpec((blk, width), lambda i: (i, 0))

    h1x, tabB = pl.pallas_call(
        _tc1_body,
        grid=grid,
        in_specs=[rows(f_in), full((f_in, hdim)), full((hdim, 16)),
                  full((hdim, 16))],
        out_specs=[rows(hdim + 16), rows(16)],
        out_shape=[jax.ShapeDtypeStruct((n, hdim + 16), jnp.float32),
                   jax.ShapeDtypeStruct((n, 16), jnp.float32)],
    )(x, W1, acatA, acatB)

    tabBp = jnp.concatenate(
        [tabB, jnp.zeros((n_pad - n, 16), jnp.float32)], axis=0)

    zf = jnp.zeros((n_pad, hdim + 16), jnp.float32)
    sc1 = _make_sc_edge_kernel(n_pad, e_pad, hdim, CHUNK1, heads)
    outp1 = sc1(packed.reshape(-1, CHUNK1), h1x, tabBp, zf)

    h2x, tab2B = pl.pallas_call(
        _tc2_body,
        grid=grid,
        in_specs=[rows(hdim + 16), rows(hdim + 16),
                  full((16, hdim)), pl.BlockSpec((1, hdim), lambda i: (0, 0)),
                  full((hdim, out_f)), full((out_f, 16)), full((out_f, 16))],
        out_specs=[rows(out_f + 16), rows(16)],
        out_shape=[jax.ShapeDtypeStruct((n, out_f + 16), jnp.float32),
                   jax.ShapeDtypeStruct((n, 16), jnp.float32)],
    )(outp1[:n], outp1[n_pad:n_pad + n], R16, b1.reshape(1, hdim),
      W2, a2A, a2B)

    tab2Bp = jnp.concatenate(
        [tab2B, jnp.zeros((n_pad - n, 16), jnp.float32)], axis=0)

    zf2 = jnp.zeros((n_pad, out_f + 16), jnp.float32)
    sc2 = _make_sc_edge_kernel(n_pad, e_pad, out_f, CHUNK2, 1)
    outp2 = sc2(packed.reshape(-1, CHUNK2), h2x, tab2Bp, zf2)

    final = _make_sc_final_kernel(n_pad, out_f, nb)
    return final(outp2, node_index, b2)
